# baseline (speedup 1.0000x reference)
import functools

import jax
import jax.numpy as jnp
from jax import lax
from jax.experimental import pallas as pl
from jax.experimental.pallas import tpu as pltpu
from jax.experimental.pallas import tpu_sc as plsc

N = 10000
E = 160000
D = 64
DP = 128
EBLK = 2000
NEB = E // EBLK
GCH = 128


def _sc_gather(table, idx_pad, ep):
    info = plsc.get_sparse_core_info()
    nw = info.num_cores * info.num_subcores
    b_per_w = ep // nw
    n_ch = b_per_w // GCH
    mesh = plsc.VectorSubcoreMesh(core_axis_name="c", subcore_axis_name="s")

    @functools.partial(
        pl.kernel,
        mesh=mesh,
        out_type=jax.ShapeDtypeStruct((ep, DP), jnp.float32),
        scratch_types=[
            pltpu.VMEM((GCH,), jnp.int32),
            pltpu.VMEM((GCH, DP), jnp.float32),
            pltpu.SemaphoreType.DMA,
        ],
    )
    def k(table_hbm, idx_hbm, out_hbm, idx_v, rows_v, sem):
        wid = lax.axis_index("s") * info.num_cores + lax.axis_index("c")
        base = wid * b_per_w

        def body(j, _):
            off = base + j * GCH
            pltpu.sync_copy(idx_hbm.at[pl.ds(off, GCH)], idx_v)
            pltpu.async_copy(table_hbm.at[idx_v], rows_v, sem).wait()
            pltpu.sync_copy(rows_v, out_hbm.at[pl.ds(off, GCH)])
            return 0

        lax.fori_loop(0, n_ch, body, 0)

    return k(table, idx_pad)


def _k_matmul(x, wt, b, rblk):
    n, kdim = x.shape
    dout = wt.shape[1]

    def body(x_ref, w_ref, b_ref, y_ref):
        y_ref[...] = (
            jnp.dot(x_ref[...], w_ref[...], preferred_element_type=jnp.float32)
            + b_ref[...]
        )

    return pl.pallas_call(
        body,
        grid=(n // rblk,),
        in_specs=[
            pl.BlockSpec((rblk, kdim), lambda i: (i, 0)),
            pl.BlockSpec((kdim, dout), lambda i: (0, 0)),
            pl.BlockSpec((1, dout), lambda i: (0, 0)),
        ],
        out_specs=pl.BlockSpec((rblk, dout), lambda i: (i, 0)),
        out_shape=jax.ShapeDtypeStruct((n, dout), jnp.float32),
    )(x, wt, b)


def _k_bn_relu(t, g, b):
    def body(t_ref, g_ref, b_ref, o_ref):
        t = t_ref[...]
        mu = jnp.mean(t, axis=0, keepdims=True)
        var = jnp.mean((t - mu) ** 2, axis=0, keepdims=True)
        r = jnp.maximum(
            g_ref[...] * (t - mu) / jnp.sqrt(var + 1e-5) + b_ref[...], 0.0
        )
        o_ref[...] = jnp.concatenate(
            [r, jnp.zeros((r.shape[0], DP - D), jnp.float32)], axis=1
        )

    return pl.pallas_call(
        body,
        out_shape=jax.ShapeDtypeStruct((t.shape[0], DP), jnp.float32),
    )(t, g.reshape(1, -1), b.reshape(1, -1))


def _k_edge_mlp(xi, xj, w1it, w1dt, b1, w2t, b2):
    h1 = w1it.shape[1]

    def body(xi_ref, xj_ref, w1i_ref, w1d_ref, b1_ref, w2_ref, b2_ref, m_ref):
        xi = xi_ref[:, :D]
        xd = xj_ref[:, :D] - xi
        z = (
            jnp.dot(xi, w1i_ref[...], preferred_element_type=jnp.float32)
            + jnp.dot(xd, w1d_ref[...], preferred_element_type=jnp.float32)
            + b1_ref[...]
        )
        z = jnp.maximum(z, 0.0)
        m_ref[...] = (
            jnp.dot(z, w2_ref[...], preferred_element_type=jnp.float32) + b2_ref[...]
        )

    return pl.pallas_call(
        body,
        grid=(NEB,),
        in_specs=[
            pl.BlockSpec((EBLK, DP), lambda i: (i, 0)),
            pl.BlockSpec((EBLK, DP), lambda i: (i, 0)),
            pl.BlockSpec((D, h1), lambda i: (0, 0)),
            pl.BlockSpec((D, h1), lambda i: (0, 0)),
            pl.BlockSpec((1, h1), lambda i: (0, 0)),
            pl.BlockSpec((h1, D), lambda i: (0, 0)),
            pl.BlockSpec((1, D), lambda i: (0, 0)),
        ],
        out_specs=pl.BlockSpec((EBLK, D), lambda i: (i, 0)),
        out_shape=jax.ShapeDtypeStruct((E, D), jnp.float32),
    )(xi, xj, w1it, w1dt, b1, w2t, b2)


def _k_segment(m, seg3, op):
    mcols = m.shape[1]

    def body(seg_ref, m_ref, acc_ref):
        pid = pl.program_id(0)

        @pl.when(pid == 0)
        def _():
            acc_ref[...] = jnp.full_like(
                acc_ref, -jnp.inf if op == "max" else 0.0
            )

        def loop(i, _):
            s = seg_ref[0, 0, i]

            @pl.when(s < N)
            def _():
                row = m_ref[pl.ds(i, 1), :D]
                old = acc_ref[pl.ds(s, 1), :]
                if op == "max":
                    acc_ref[pl.ds(s, 1), :] = jnp.maximum(old, row)
                else:
                    acc_ref[pl.ds(s, 1), :] = old + row

            return 0

        lax.fori_loop(0, EBLK, loop, 0)

        if op == "max":

            @pl.when(pid == NEB - 1)
            def _():
                a = acc_ref[...]
                acc_ref[...] = jnp.where(jnp.isfinite(a), a, 0.0)

    return pl.pallas_call(
        body,
        grid=(NEB,),
        in_specs=[
            pl.BlockSpec((1, 1, EBLK), lambda i: (i, 0, 0), memory_space=pltpu.SMEM),
            pl.BlockSpec((EBLK, mcols), lambda i: (i, 0)),
        ],
        out_specs=pl.BlockSpec((N, D), lambda i: (0, 0)),
        out_shape=jax.ShapeDtypeStruct((N, D), jnp.float32),
    )(seg3, m)


def _k_segcount(seg3):
    def body(seg_ref, acc_ref):
        pid = pl.program_id(0)

        @pl.when(pid == 0)
        def _():
            acc_ref[...] = jnp.zeros_like(acc_ref)

        def loop(i, _):
            s = seg_ref[0, 0, i]

            @pl.when(s < N)
            def _():
                acc_ref[pl.ds(s, 1), :] = acc_ref[pl.ds(s, 1), :] + 1.0

            return 0

        lax.fori_loop(0, EBLK, loop, 0)

    return pl.pallas_call(
        body,
        grid=(NEB,),
        in_specs=[
            pl.BlockSpec((1, 1, EBLK), lambda i: (i, 0, 0), memory_space=pltpu.SMEM),
        ],
        out_specs=pl.BlockSpec((N, D), lambda i: (0, 0)),
        out_shape=jax.ShapeDtypeStruct((N, D), jnp.float32),
    )(seg3)


def _k_sage(s, cnt, t, wlt, bl, wrt):
    dout = wlt.shape[1]

    def body(s_ref, c_ref, t_ref, wl_ref, bl_ref, wr_ref, o_ref):
        agg = s_ref[...] / jnp.maximum(c_ref[...], 1.0)
        o_ref[...] = (
            jnp.dot(agg, wl_ref[...], preferred_element_type=jnp.float32)
            + bl_ref[...]
            + jnp.dot(t_ref[:, :D], wr_ref[...], preferred_element_type=jnp.float32)
        )

    return pl.pallas_call(
        body,
        out_shape=jax.ShapeDtypeStruct((N, dout), jnp.float32),
    )(s, cnt, t, wlt, bl.reshape(1, -1), wrt)


def _k_final(outs):
    def body(o_ref, y_ref):
        y_ref[...] = jax.nn.sigmoid(
            jnp.sum(o_ref[...], axis=1, keepdims=True)
        )

    return pl.pallas_call(
        body,
        out_shape=jax.ShapeDtypeStruct((N, 1), jnp.float32),
    )(outs)


def kernel(x, edge_index, edge_attr, params):
    p = params
    fd = 1024
    src = edge_index[0].astype(jnp.int32)
    dst = edge_index[1].astype(jnp.int32)

    ep = 163840
    pad = ep - E
    src_p = jnp.concatenate([src, jnp.zeros((pad,), jnp.int32)])
    dst_p = jnp.concatenate([dst, jnp.zeros((pad,), jnp.int32)])

    m1 = edge_attr >= 0
    m2 = edge_attr <= 0
    seg_m1 = jnp.where(m1, dst, N).reshape(NEB, 1, EBLK)
    seg_m2 = jnp.where(m2, dst, N).reshape(NEB, 1, EBLK)
    seg_all = dst.reshape(NEB, 1, EBLK)
    segs = {"m1": seg_m1, "m2": seg_m2, "mall": seg_all}

    w011a = p["l011_W"][:, : fd // 2]
    w011b = p["l011_W"][:, fd // 2 :]
    wbig = jnp.concatenate([p["l012_W"], w011a, w011b @ p["spf_W"]], axis=1)
    bbig = (p["l011_b"] + p["l012_b"] + w011b @ p["spf_b"]).reshape(1, -1)

    y = _k_matmul(x, wbig.T, bbig, rblk=2000)
    h = _k_bn_relu(y, p["bn01_g"], p["bn01_b"])

    xi = _sc_gather(h, dst_p, ep)
    xj = _sc_gather(h, src_p, ep)

    msgs = {}
    for fam in ["ec11", "ec12", "ec13", "ecO", "ecO2"]:
        w1, b1 = p[fam + "_W1"], p[fam + "_b1"]
        w2, b2 = p[fam + "_W2"], p[fam + "_b2"]
        msgs[fam] = _k_edge_mlp(
            xi, xj,
            w1[:, :D].T, w1[:, D:].T, b1.reshape(1, -1),
            w2.T, b2.reshape(1, -1),
        )

    cnts = {k: _k_segcount(v) for k, v in segs.items()}

    branches = [
        ("ec11", "m1", "bn11", "m1", "s31", "m1"),
        ("ec12", "m2", "bn12", "m2", "s32", "m2"),
        ("ec13", "mall", "bn13", "mall", "s33", "mall"),
        ("ecO", "m1", "bn11", "m1", "s31", "m1"),
        ("ecO", "m2", "bn12", "m1", "s32", "m2"),
        ("ecO", "mall", "bn13", "m1", "s33", "mall"),
        ("ecO2", "m1", "bn11", "m1", "s31", "m1"),
        ("ecO2", "m2", "bn12", "m1", "s32", "m2"),
        ("ecO2", "mall", "bn13", "m1", "s33", "mall"),
    ]

    outs = []
    for fam, ec_m, bn, mid_m, out_w, out_m in branches:
        t = _k_segment(msgs[fam], segs[ec_m], "max")
        t = _k_bn_relu(t, p[bn + "_g"], p[bn + "_b"])
        tg = _sc_gather(t, src_p, ep)
        s_mid = _k_segment(tg, segs[mid_m], "sum")
        u = _k_sage(s_mid, cnts[mid_m], t, p["s21_Wl"].T, p["s21_bl"], p["s21_Wr"].T)
        u = _k_bn_relu(u, p["bn21_g"], p["bn21_b"])
        ug = _sc_gather(u, src_p, ep)
        s_out = _k_segment(ug, segs[out_m], "sum")
        outs.append(
            _k_sage(
                s_out, cnts[out_m], u,
                p[out_w + "_Wl"].T, p[out_w + "_bl"], p[out_w + "_Wr"].T,
            )
        )

    return _k_final(jnp.concatenate(outs, axis=1))


# speedup vs baseline: 2.5108x; 2.5108x over previous
import functools

import jax
import jax.numpy as jnp
from jax import lax
from jax.experimental import pallas as pl
from jax.experimental.pallas import tpu as pltpu
from jax.experimental.pallas import tpu_sc as plsc

N = 10000
E = 160000
D = 64
DP = 128
EBLK = 2000
NEB = E // EBLK
GCH = 128


def _sc_gather(table, idx_pad, ep):
    info = plsc.get_sparse_core_info()
    nw = info.num_cores * info.num_subcores
    b_per_w = ep // nw
    n_ch = b_per_w // GCH
    mesh = plsc.VectorSubcoreMesh(core_axis_name="c", subcore_axis_name="s")

    @functools.partial(
        pl.kernel,
        mesh=mesh,
        out_type=jax.ShapeDtypeStruct((ep, DP), jnp.float32),
        scratch_types=[
            pltpu.VMEM((GCH,), jnp.int32),
            pltpu.VMEM((GCH, DP), jnp.float32),
            pltpu.SemaphoreType.DMA,
        ],
    )
    def k(table_hbm, idx_hbm, out_hbm, idx_v, rows_v, sem):
        wid = lax.axis_index("s") * info.num_cores + lax.axis_index("c")
        base = wid * b_per_w

        def body(j, _):
            off = base + j * GCH
            pltpu.sync_copy(idx_hbm.at[pl.ds(off, GCH)], idx_v)
            pltpu.async_copy(table_hbm.at[idx_v], rows_v, sem).wait()
            pltpu.sync_copy(rows_v, out_hbm.at[pl.ds(off, GCH)])
            return 0

        lax.fori_loop(0, n_ch, body, 0)

    return k(table, idx_pad)


NR = 10240


def _sc_scatter_add(vals, seg_pad, ep, count=False):
    info = plsc.get_sparse_core_info()
    nc, ns = info.num_cores, info.num_subcores
    b_per_w = ep // (nc * ns)
    n_ch = b_per_w // GCH
    rows_per_sub = NR // ns
    mesh = plsc.VectorSubcoreMesh(core_axis_name="c", subcore_axis_name="s")

    zeros = jnp.zeros((rows_per_sub, DP), jnp.float32)
    ones = jnp.ones((GCH, DP), jnp.float32)

    @functools.partial(
        pl.kernel,
        mesh=mesh,
        out_type=jax.ShapeDtypeStruct((nc, NR, DP), jnp.float32),
        scratch_types=[
            pltpu.VMEM((GCH,), jnp.int32),
            pltpu.VMEM((GCH, DP), jnp.float32),
            pltpu.VMEM_SHARED((NR, DP), jnp.float32),
        ],
    )
    def k(vals_hbm, seg_hbm, zeros_hbm, ones_hbm, out_hbm, idx_v, rows_v, acc_sh):
        c = lax.axis_index("c")
        sid = lax.axis_index("s")
        wid = sid * nc + c
        base = wid * b_per_w

        pltpu.sync_copy(zeros_hbm, acc_sh.at[pl.ds(sid * rows_per_sub, rows_per_sub)])
        plsc.subcore_barrier()

        if count:
            pltpu.sync_copy(ones_hbm, rows_v)

        def body(j, _):
            off = base + j * GCH
            pltpu.sync_copy(seg_hbm.at[pl.ds(off, GCH)], idx_v)
            if not count:
                pltpu.sync_copy(vals_hbm.at[pl.ds(off, GCH)], rows_v)
            pltpu.sync_copy(rows_v, acc_sh.at[idx_v], add=True)
            return 0

        lax.fori_loop(0, n_ch, body, 0)
        plsc.subcore_barrier()
        pltpu.sync_copy(
            acc_sh.at[pl.ds(sid * rows_per_sub, rows_per_sub)],
            out_hbm.at[c, pl.ds(sid * rows_per_sub, rows_per_sub)],
        )

    return k(vals, seg_pad, zeros, ones)


def _k_matmul(x, wt, b, rblk):
    n, kdim = x.shape
    dout = wt.shape[1]

    def body(x_ref, w_ref, b_ref, y_ref):
        y_ref[...] = (
            jnp.dot(x_ref[...], w_ref[...], preferred_element_type=jnp.float32)
            + b_ref[...]
        )

    return pl.pallas_call(
        body,
        grid=(n // rblk,),
        in_specs=[
            pl.BlockSpec((rblk, kdim), lambda i: (i, 0)),
            pl.BlockSpec((kdim, dout), lambda i: (0, 0)),
            pl.BlockSpec((1, dout), lambda i: (0, 0)),
        ],
        out_specs=pl.BlockSpec((rblk, dout), lambda i: (i, 0)),
        out_shape=jax.ShapeDtypeStruct((n, dout), jnp.float32),
    )(x, wt, b)


def _k_bn_relu(t, g, b):
    def body(t_ref, g_ref, b_ref, o_ref):
        t = t_ref[...]
        mu = jnp.mean(t, axis=0, keepdims=True)
        var = jnp.mean((t - mu) ** 2, axis=0, keepdims=True)
        r = jnp.maximum(
            g_ref[...] * (t - mu) / jnp.sqrt(var + 1e-5) + b_ref[...], 0.0
        )
        o_ref[...] = jnp.concatenate(
            [r, jnp.zeros((r.shape[0], DP - D), jnp.float32)], axis=1
        )

    return pl.pallas_call(
        body,
        out_shape=jax.ShapeDtypeStruct((t.shape[0], DP), jnp.float32),
    )(t, g.reshape(1, -1), b.reshape(1, -1))


def _k_edge_mlp(xi, xj, w1it, w1dt, b1, w2t, b2):
    h1 = w1it.shape[1]

    def body(xi_ref, xj_ref, w1i_ref, w1d_ref, b1_ref, w2_ref, b2_ref, m_ref):
        xi = xi_ref[:, :D]
        xd = xj_ref[:, :D] - xi
        z = (
            jnp.dot(xi, w1i_ref[...], preferred_element_type=jnp.float32)
            + jnp.dot(xd, w1d_ref[...], preferred_element_type=jnp.float32)
            + b1_ref[...]
        )
        z = jnp.maximum(z, 0.0)
        m_ref[...] = (
            jnp.dot(z, w2_ref[...], preferred_element_type=jnp.float32) + b2_ref[...]
        )

    return pl.pallas_call(
        body,
        grid=(NEB,),
        in_specs=[
            pl.BlockSpec((EBLK, DP), lambda i: (i, 0)),
            pl.BlockSpec((EBLK, DP), lambda i: (i, 0)),
            pl.BlockSpec((D, h1), lambda i: (0, 0)),
            pl.BlockSpec((D, h1), lambda i: (0, 0)),
            pl.BlockSpec((1, h1), lambda i: (0, 0)),
            pl.BlockSpec((h1, D), lambda i: (0, 0)),
            pl.BlockSpec((1, D), lambda i: (0, 0)),
        ],
        out_specs=pl.BlockSpec((EBLK, D), lambda i: (i, 0)),
        out_shape=jax.ShapeDtypeStruct((E, D), jnp.float32),
    )(xi, xj, w1it, w1dt, b1, w2t, b2)


def _k_segment(m, seg3, op):
    mcols = m.shape[1]

    def body(seg_ref, m_ref, acc_ref):
        pid = pl.program_id(0)

        @pl.when(pid == 0)
        def _():
            acc_ref[...] = jnp.full_like(
                acc_ref, -jnp.inf if op == "max" else 0.0
            )

        def loop(i, _):
            s = seg_ref[0, 0, i]

            @pl.when(s < N)
            def _():
                row = m_ref[pl.ds(i, 1), :D]
                old = acc_ref[pl.ds(s, 1), :]
                if op == "max":
                    acc_ref[pl.ds(s, 1), :] = jnp.maximum(old, row)
                else:
                    acc_ref[pl.ds(s, 1), :] = old + row

            return 0

        lax.fori_loop(0, EBLK, loop, 0)

        if op == "max":

            @pl.when(pid == NEB - 1)
            def _():
                a = acc_ref[...]
                acc_ref[...] = jnp.where(jnp.isfinite(a), a, 0.0)

    return pl.pallas_call(
        body,
        grid=(NEB,),
        in_specs=[
            pl.BlockSpec((1, 1, EBLK), lambda i: (i, 0, 0), memory_space=pltpu.SMEM),
            pl.BlockSpec((EBLK, mcols), lambda i: (i, 0)),
        ],
        out_specs=pl.BlockSpec((N, D), lambda i: (0, 0)),
        out_shape=jax.ShapeDtypeStruct((N, D), jnp.float32),
    )(seg3, m)


def _k_segcount(seg3):
    def body(seg_ref, acc_ref):
        pid = pl.program_id(0)

        @pl.when(pid == 0)
        def _():
            acc_ref[...] = jnp.zeros_like(acc_ref)

        def loop(i, _):
            s = seg_ref[0, 0, i]

            @pl.when(s < N)
            def _():
                acc_ref[pl.ds(s, 1), :] = acc_ref[pl.ds(s, 1), :] + 1.0

            return 0

        lax.fori_loop(0, EBLK, loop, 0)

    return pl.pallas_call(
        body,
        grid=(NEB,),
        in_specs=[
            pl.BlockSpec((1, 1, EBLK), lambda i: (i, 0, 0), memory_space=pltpu.SMEM),
        ],
        out_specs=pl.BlockSpec((N, D), lambda i: (0, 0)),
        out_shape=jax.ShapeDtypeStruct((N, D), jnp.float32),
    )(seg3)


def _k_sage(s2, cnt2, t, wlt, bl, wrt):
    dout = wlt.shape[1]

    def body(s_ref, c_ref, t_ref, wl_ref, bl_ref, wr_ref, o_ref):
        s = s_ref[0, :N, :D] + s_ref[1, :N, :D]
        c = c_ref[0, :N, :D] + c_ref[1, :N, :D]
        agg = s / jnp.maximum(c, 1.0)
        o_ref[...] = (
            jnp.dot(agg, wl_ref[...], preferred_element_type=jnp.float32)
            + bl_ref[...]
            + jnp.dot(t_ref[:, :D], wr_ref[...], preferred_element_type=jnp.float32)
        )

    return pl.pallas_call(
        body,
        out_shape=jax.ShapeDtypeStruct((N, dout), jnp.float32),
    )(s2, cnt2, t, wlt, bl.reshape(1, -1), wrt)


def _k_final(outs):
    def body(o_ref, y_ref):
        y_ref[...] = jax.nn.sigmoid(
            jnp.sum(o_ref[...], axis=1, keepdims=True)
        )

    return pl.pallas_call(
        body,
        out_shape=jax.ShapeDtypeStruct((N, 1), jnp.float32),
    )(outs)


def kernel(x, edge_index, edge_attr, params):
    p = params
    fd = 1024
    src = edge_index[0].astype(jnp.int32)
    dst = edge_index[1].astype(jnp.int32)

    ep = 163840
    pad = ep - E
    src_p = jnp.concatenate([src, jnp.zeros((pad,), jnp.int32)])
    dst_p = jnp.concatenate([dst, jnp.zeros((pad,), jnp.int32)])

    m1 = edge_attr >= 0
    m2 = edge_attr <= 0
    seg_m1 = jnp.where(m1, dst, N)
    seg_m2 = jnp.where(m2, dst, N)
    segs = {
        "m1": seg_m1.reshape(NEB, 1, EBLK),
        "m2": seg_m2.reshape(NEB, 1, EBLK),
        "mall": dst.reshape(NEB, 1, EBLK),
    }
    pad_ids = jnp.full((pad,), N, jnp.int32)
    segp = {
        "m1": jnp.concatenate([seg_m1, pad_ids]),
        "m2": jnp.concatenate([seg_m2, pad_ids]),
        "mall": jnp.concatenate([dst, pad_ids]),
    }

    w011a = p["l011_W"][:, : fd // 2]
    w011b = p["l011_W"][:, fd // 2 :]
    wbig = jnp.concatenate([p["l012_W"], w011a, w011b @ p["spf_W"]], axis=1)
    bbig = (p["l011_b"] + p["l012_b"] + w011b @ p["spf_b"]).reshape(1, -1)

    y = _k_matmul(x, wbig.T, bbig, rblk=2000)
    h = _k_bn_relu(y, p["bn01_g"], p["bn01_b"])

    xi = _sc_gather(h, dst_p, ep)
    xj = _sc_gather(h, src_p, ep)

    msgs = {}
    for fam in ["ec11", "ec12", "ec13", "ecO", "ecO2"]:
        w1, b1 = p[fam + "_W1"], p[fam + "_b1"]
        w2, b2 = p[fam + "_W2"], p[fam + "_b2"]
        msgs[fam] = _k_edge_mlp(
            xi, xj,
            w1[:, :D].T, w1[:, D:].T, b1.reshape(1, -1),
            w2.T, b2.reshape(1, -1),
        )

    cnts = {k: _sc_scatter_add(xi, v, ep, count=True) for k, v in segp.items()}

    branches = [
        ("ec11", "m1", "bn11", "m1", "s31", "m1"),
        ("ec12", "m2", "bn12", "m2", "s32", "m2"),
        ("ec13", "mall", "bn13", "mall", "s33", "mall"),
        ("ecO", "m1", "bn11", "m1", "s31", "m1"),
        ("ecO", "m2", "bn12", "m1", "s32", "m2"),
        ("ecO", "mall", "bn13", "m1", "s33", "mall"),
        ("ecO2", "m1", "bn11", "m1", "s31", "m1"),
        ("ecO2", "m2", "bn12", "m1", "s32", "m2"),
        ("ecO2", "mall", "bn13", "m1", "s33", "mall"),
    ]

    outs = []
    for fam, ec_m, bn, mid_m, out_w, out_m in branches:
        t = _k_segment(msgs[fam], segs[ec_m], "max")
        t = _k_bn_relu(t, p[bn + "_g"], p[bn + "_b"])
        tg = _sc_gather(t, src_p, ep)
        s_mid = _sc_scatter_add(tg, segp[mid_m], ep)
        u = _k_sage(s_mid, cnts[mid_m], t, p["s21_Wl"].T, p["s21_bl"], p["s21_Wr"].T)
        u = _k_bn_relu(u, p["bn21_g"], p["bn21_b"])
        ug = _sc_gather(u, src_p, ep)
        s_out = _sc_scatter_add(ug, segp[out_m], ep)
        outs.append(
            _k_sage(
                s_out, cnts[out_m], u,
                p[out_w + "_Wl"].T, p[out_w + "_bl"], p[out_w + "_Wr"].T,
            )
        )

    return _k_final(jnp.concatenate(outs, axis=1))


# speedup vs baseline: 4.3526x; 1.7336x over previous
import functools

import jax
import jax.numpy as jnp
from jax import lax
from jax.experimental import pallas as pl
from jax.experimental.pallas import tpu as pltpu
from jax.experimental.pallas import tpu_sc as plsc

N = 10000
E = 160000
D = 64
DP = 128
EBLK = 2000
NEB = E // EBLK
GCH = 128


def _sc_gather(table, idx_pad, ep):
    info = plsc.get_sparse_core_info()
    nw = info.num_cores * info.num_subcores
    b_per_w = ep // nw
    n_ch = b_per_w // GCH
    mesh = plsc.VectorSubcoreMesh(core_axis_name="c", subcore_axis_name="s")

    @functools.partial(
        pl.kernel,
        mesh=mesh,
        out_type=jax.ShapeDtypeStruct((ep, DP), jnp.float32),
        scratch_types=[
            pltpu.VMEM((GCH,), jnp.int32),
            pltpu.VMEM((GCH, DP), jnp.float32),
            pltpu.SemaphoreType.DMA,
        ],
    )
    def k(table_hbm, idx_hbm, out_hbm, idx_v, rows_v, sem):
        wid = lax.axis_index("s") * info.num_cores + lax.axis_index("c")
        base = wid * b_per_w

        def body(j, _):
            off = base + j * GCH
            pltpu.sync_copy(idx_hbm.at[pl.ds(off, GCH)], idx_v)
            pltpu.async_copy(table_hbm.at[idx_v], rows_v, sem).wait()
            pltpu.sync_copy(rows_v, out_hbm.at[pl.ds(off, GCH)])
            return 0

        lax.fori_loop(0, n_ch, body, 0)

    return k(table, idx_pad)


NR = 10240


def _sc_scatter_add(vals, seg_pad, ep, count=False):
    info = plsc.get_sparse_core_info()
    nc, ns = info.num_cores, info.num_subcores
    b_per_w = ep // (nc * ns)
    n_ch = b_per_w // GCH
    rows_per_sub = NR // ns
    mesh = plsc.VectorSubcoreMesh(core_axis_name="c", subcore_axis_name="s")

    zeros = jnp.zeros((rows_per_sub, DP), jnp.float32)
    ones = jnp.ones((GCH, DP), jnp.float32)

    @functools.partial(
        pl.kernel,
        mesh=mesh,
        out_type=jax.ShapeDtypeStruct((nc, NR, DP), jnp.float32),
        scratch_types=[
            pltpu.VMEM((GCH,), jnp.int32),
            pltpu.VMEM((GCH, DP), jnp.float32),
            pltpu.VMEM_SHARED((NR, DP), jnp.float32),
        ],
    )
    def k(vals_hbm, seg_hbm, zeros_hbm, ones_hbm, out_hbm, idx_v, rows_v, acc_sh):
        c = lax.axis_index("c")
        sid = lax.axis_index("s")
        wid = sid * nc + c
        base = wid * b_per_w

        pltpu.sync_copy(zeros_hbm, acc_sh.at[pl.ds(sid * rows_per_sub, rows_per_sub)])
        plsc.subcore_barrier()

        if count:
            pltpu.sync_copy(ones_hbm, rows_v)

        def body(j, _):
            off = base + j * GCH
            pltpu.sync_copy(seg_hbm.at[pl.ds(off, GCH)], idx_v)
            if not count:
                pltpu.sync_copy(vals_hbm.at[pl.ds(off, GCH)], rows_v)
            pltpu.sync_copy(rows_v, acc_sh.at[idx_v], add=True)
            return 0

        lax.fori_loop(0, n_ch, body, 0)
        plsc.subcore_barrier()
        pltpu.sync_copy(
            acc_sh.at[pl.ds(sid * rows_per_sub, rows_per_sub)],
            out_hbm.at[c, pl.ds(sid * rows_per_sub, rows_per_sub)],
        )

    return k(vals, seg_pad, zeros, ones)


def _k_matmul(x, wt, b, rblk):
    n, kdim = x.shape
    dout = wt.shape[1]

    def body(x_ref, w_ref, b_ref, y_ref):
        y_ref[...] = (
            jnp.dot(x_ref[...], w_ref[...], preferred_element_type=jnp.float32)
            + b_ref[...]
        )

    return pl.pallas_call(
        body,
        grid=(n // rblk,),
        in_specs=[
            pl.BlockSpec((rblk, kdim), lambda i: (i, 0)),
            pl.BlockSpec((kdim, dout), lambda i: (0, 0)),
            pl.BlockSpec((1, dout), lambda i: (0, 0)),
        ],
        out_specs=pl.BlockSpec((rblk, dout), lambda i: (i, 0)),
        out_shape=jax.ShapeDtypeStruct((n, dout), jnp.float32),
    )(x, wt, b)


def _k_bn_relu(t, g, b):
    cols = t.shape[1]
    ocols = -(-cols // DP) * DP

    def body(t_ref, g_ref, b_ref, o_ref):
        t = t_ref[...]
        mu = jnp.mean(t, axis=0, keepdims=True)
        var = jnp.mean((t - mu) ** 2, axis=0, keepdims=True)
        r = jnp.maximum(
            g_ref[...] * (t - mu) / jnp.sqrt(var + 1e-5) + b_ref[...], 0.0
        )
        o_ref[...] = jnp.concatenate(
            [r, jnp.zeros((r.shape[0], ocols - cols), jnp.float32)], axis=1
        )

    return pl.pallas_call(
        body,
        out_shape=jax.ShapeDtypeStruct((t.shape[0], ocols), jnp.float32),
    )(t, g.reshape(1, -1), b.reshape(1, -1))


def _k_edge_mlp(xi, xj, w1it, w1dt, b1, w2t, b2):
    h1 = w1it.shape[1]

    def body(xi_ref, xj_ref, w1i_ref, w1d_ref, b1_ref, w2_ref, b2_ref, m_ref):
        xi = xi_ref[:, :D]
        xd = xj_ref[:, :D] - xi
        z = (
            jnp.dot(xi, w1i_ref[...], preferred_element_type=jnp.float32)
            + jnp.dot(xd, w1d_ref[...], preferred_element_type=jnp.float32)
            + b1_ref[...]
        )
        z = jnp.maximum(z, 0.0)
        m_ref[...] = (
            jnp.dot(z, w2_ref[...], preferred_element_type=jnp.float32) + b2_ref[...]
        )

    return pl.pallas_call(
        body,
        grid=(NEB,),
        in_specs=[
            pl.BlockSpec((EBLK, DP), lambda i: (i, 0)),
            pl.BlockSpec((EBLK, DP), lambda i: (i, 0)),
            pl.BlockSpec((D, h1), lambda i: (0, 0)),
            pl.BlockSpec((D, h1), lambda i: (0, 0)),
            pl.BlockSpec((1, h1), lambda i: (0, 0)),
            pl.BlockSpec((h1, D), lambda i: (0, 0)),
            pl.BlockSpec((1, D), lambda i: (0, 0)),
        ],
        out_specs=pl.BlockSpec((EBLK, D), lambda i: (i, 0)),
        out_shape=jax.ShapeDtypeStruct((E, D), jnp.float32),
    )(xi, xj, w1it, w1dt, b1, w2t, b2)


def _k_segment(m, seg3, op):
    mcols = m.shape[1]
    ocols = mcols if op == "max" else D

    def body(seg_ref, m_ref, acc_ref):
        pid = pl.program_id(0)

        @pl.when(pid == 0)
        def _():
            acc_ref[...] = jnp.full_like(
                acc_ref, -jnp.inf if op == "max" else 0.0
            )

        def loop(i, _):
            s = seg_ref[0, 0, i]

            @pl.when(s < N)
            def _():
                row = m_ref[pl.ds(i, 1), :ocols]
                old = acc_ref[pl.ds(s, 1), :]
                if op == "max":
                    acc_ref[pl.ds(s, 1), :] = jnp.maximum(old, row)
                else:
                    acc_ref[pl.ds(s, 1), :] = old + row

            return 0

        lax.fori_loop(0, EBLK, loop, 0)

        if op == "max":

            @pl.when(pid == NEB - 1)
            def _():
                a = acc_ref[...]
                acc_ref[...] = jnp.where(jnp.isfinite(a), a, 0.0)

    return pl.pallas_call(
        body,
        grid=(NEB,),
        in_specs=[
            pl.BlockSpec((1, 1, EBLK), lambda i: (i, 0, 0), memory_space=pltpu.SMEM),
            pl.BlockSpec((EBLK, mcols), lambda i: (i, 0)),
        ],
        out_specs=pl.BlockSpec((N, ocols), lambda i: (0, 0)),
        out_shape=jax.ShapeDtypeStruct((N, ocols), jnp.float32),
    )(seg3, m)


def _k_segcount(seg3):
    def body(seg_ref, acc_ref):
        pid = pl.program_id(0)

        @pl.when(pid == 0)
        def _():
            acc_ref[...] = jnp.zeros_like(acc_ref)

        def loop(i, _):
            s = seg_ref[0, 0, i]

            @pl.when(s < N)
            def _():
                acc_ref[pl.ds(s, 1), :] = acc_ref[pl.ds(s, 1), :] + 1.0

            return 0

        lax.fori_loop(0, EBLK, loop, 0)

    return pl.pallas_call(
        body,
        grid=(NEB,),
        in_specs=[
            pl.BlockSpec((1, 1, EBLK), lambda i: (i, 0, 0), memory_space=pltpu.SMEM),
        ],
        out_specs=pl.BlockSpec((N, D), lambda i: (0, 0)),
        out_shape=jax.ShapeDtypeStruct((N, D), jnp.float32),
    )(seg3)


def _k_sage(s2, cnt2, t, wlt, bl, wrt):
    dout = wlt.shape[1]

    def body(s_ref, c_ref, t_ref, wl_ref, bl_ref, wr_ref, o_ref):
        s = s_ref[0, :N, :D] + s_ref[1, :N, :D]
        c = c_ref[0, :N, :D] + c_ref[1, :N, :D]
        agg = s / jnp.maximum(c, 1.0)
        o_ref[...] = (
            jnp.dot(agg, wl_ref[...], preferred_element_type=jnp.float32)
            + bl_ref[...]
            + jnp.dot(t_ref[:, :D], wr_ref[...], preferred_element_type=jnp.float32)
        )

    return pl.pallas_call(
        body,
        out_shape=jax.ShapeDtypeStruct((N, dout), jnp.float32),
    )(s2, cnt2, t, wlt, bl.reshape(1, -1), wrt)


def _k_final(outs):
    def body(o_ref, y_ref):
        y_ref[...] = jax.nn.sigmoid(
            jnp.sum(o_ref[...], axis=1, keepdims=True)
        )

    return pl.pallas_call(
        body,
        out_shape=jax.ShapeDtypeStruct((N, 1), jnp.float32),
    )(outs)


def kernel(x, edge_index, edge_attr, params):
    p = params
    fd = 1024
    src = edge_index[0].astype(jnp.int32)
    dst = edge_index[1].astype(jnp.int32)

    ep = 163840
    pad = ep - E
    src_p = jnp.concatenate([src, jnp.zeros((pad,), jnp.int32)])
    dst_p = jnp.concatenate([dst, jnp.zeros((pad,), jnp.int32)])

    m1 = edge_attr >= 0
    m2 = edge_attr <= 0
    seg_m1 = jnp.where(m1, dst, N)
    seg_m2 = jnp.where(m2, dst, N)
    segs = {
        "m1": seg_m1.reshape(NEB, 1, EBLK),
        "m2": seg_m2.reshape(NEB, 1, EBLK),
        "mall": dst.reshape(NEB, 1, EBLK),
    }
    pad_ids = jnp.full((pad,), N, jnp.int32)
    segp = {
        "m1": jnp.concatenate([seg_m1, pad_ids]),
        "m2": jnp.concatenate([seg_m2, pad_ids]),
        "mall": jnp.concatenate([dst, pad_ids]),
    }

    w011a = p["l011_W"][:, : fd // 2]
    w011b = p["l011_W"][:, fd // 2 :]
    wbig = jnp.concatenate([p["l012_W"], w011a, w011b @ p["spf_W"]], axis=1)
    bbig = (p["l011_b"] + p["l012_b"] + w011b @ p["spf_b"]).reshape(1, -1)

    y = _k_matmul(x, wbig.T, bbig, rblk=2000)
    h = _k_bn_relu(y, p["bn01_g"], p["bn01_b"])

    xi = _sc_gather(h, dst_p, ep)
    xj = _sc_gather(h, src_p, ep)

    msgs = {}
    for fam in ["ec11", "ec12", "ec13", "ecO", "ecO2"]:
        w1, b1 = p[fam + "_W1"], p[fam + "_b1"]
        w2, b2 = p[fam + "_W2"], p[fam + "_b2"]
        msgs[fam] = _k_edge_mlp(
            xi, xj,
            w1[:, :D].T, w1[:, D:].T, b1.reshape(1, -1),
            w2.T, b2.reshape(1, -1),
        )

    cnts = {k: _sc_scatter_add(xi, v, ep, count=True) for k, v in segp.items()}

    branches = [
        ("ec11", "m1", "bn11", "m1", "s31", "m1"),
        ("ec12", "m2", "bn12", "m2", "s32", "m2"),
        ("ec13", "mall", "bn13", "mall", "s33", "mall"),
        ("ecO", "m1", "bn11", "m1", "s31", "m1"),
        ("ecO", "m2", "bn12", "m1", "s32", "m2"),
        ("ecO", "mall", "bn13", "m1", "s33", "mall"),
        ("ecO2", "m1", "bn11", "m1", "s31", "m1"),
        ("ecO2", "m2", "bn12", "m1", "s32", "m2"),
        ("ecO2", "mall", "bn13", "m1", "s33", "mall"),
    ]

    fams_by_mask = {
        "m1": ["ec11", "ecO", "ecO2"],
        "m2": ["ec12", "ecO", "ecO2"],
        "mall": ["ec13", "ecO", "ecO2"],
    }
    bn_by_mask = {"m1": "bn11", "m2": "bn12", "mall": "bn13"}
    tbn = {}
    for mk, fams in fams_by_mask.items():
        mg = jnp.concatenate([msgs[f] for f in fams], axis=1)
        tm = _k_segment(mg, segs[mk], "max")
        bn = bn_by_mask[mk]
        tbn[mk] = _k_bn_relu(tm, jnp.tile(p[bn + "_g"], 3), jnp.tile(p[bn + "_b"], 3))

    outs = []
    for fam, ec_m, bn, mid_m, out_w, out_m in branches:
        kk = fams_by_mask[ec_m].index(fam)
        t = jnp.pad(tbn[ec_m][:, kk * D:(kk + 1) * D], ((0, 0), (0, DP - D)))
        tg = _sc_gather(t, src_p, ep)
        s_mid = _sc_scatter_add(tg, segp[mid_m], ep)
        u = _k_sage(s_mid, cnts[mid_m], t, p["s21_Wl"].T, p["s21_bl"], p["s21_Wr"].T)
        u = _k_bn_relu(u, p["bn21_g"], p["bn21_b"])
        ug = _sc_gather(u, src_p, ep)
        s_out = _sc_scatter_add(ug, segp[out_m], ep)
        outs.append(
            _k_sage(
                s_out, cnts[out_m], u,
                p[out_w + "_Wl"].T, p[out_w + "_bl"], p[out_w + "_Wr"].T,
            )
        )

    return _k_final(jnp.concatenate(outs, axis=1))
